# baseline (speedup 1.0000x reference)
import functools

import jax
import jax.numpy as jnp
from jax import lax
from jax.experimental import pallas as pl
from jax.experimental.pallas import tpu as pltpu
from jax.experimental.pallas import tpu_sc as plsc

N_NODES = 10000
NC, NS = 2, 16
N_ACC = 10112
RPT = N_ACC // NS
CHUNK = 128
K0 = 64
K1 = 96
DEG_W = 128

_SC_MESH = plsc.VectorSubcoreMesh(core_axis_name="c", subcore_axis_name="s")


NBUF = 2


def _make_agg(n_chunks, d):
    @functools.partial(
        pl.kernel,
        mesh=_SC_MESH,
        out_type=jax.ShapeDtypeStruct((NC, N_ACC, d), jnp.float32),
        scratch_types=[
            pltpu.VMEM((n_chunks, CHUNK), jnp.int32),
            pltpu.VMEM((n_chunks, CHUNK), jnp.int32),
            pltpu.VMEM((CHUNK, d), jnp.float32),
            pltpu.VMEM_SHARED((N_ACC, d), jnp.float32),
        ],
    )
    def agg(src_hbm, dst_hbm, g_hbm, zeros_hbm, out_hbm, src_v, dst_v, rows_v,
            acc):
        c = lax.axis_index("c")
        s = lax.axis_index("s")
        pltpu.sync_copy(src_hbm.at[c, s], src_v)
        pltpu.sync_copy(dst_hbm.at[c, s], dst_v)
        pltpu.sync_copy(zeros_hbm, acc.at[pl.ds(s * RPT, RPT)])
        plsc.subcore_barrier()

        def step(j, carry):
            pltpu.sync_copy(g_hbm.at[src_v.at[j]], rows_v)
            pltpu.sync_copy(rows_v, acc.at[dst_v.at[j]], add=True)
            return carry

        lax.fori_loop(0, lax.select(c == 0, K0, K1), step, 0)
        plsc.subcore_barrier()
        pltpu.sync_copy(acc.at[pl.ds(s * RPT, RPT)],
                        out_hbm.at[c, pl.ds(s * RPT, RPT)])

    return agg


def _make_deg(n_chunks):
    @functools.partial(
        pl.kernel,
        mesh=_SC_MESH,
        out_type=jax.ShapeDtypeStruct((NC, N_ACC, DEG_W), jnp.float32),
        scratch_types=[
            pltpu.VMEM((n_chunks, CHUNK), jnp.int32),
            pltpu.VMEM((CHUNK, DEG_W), jnp.float32),
            pltpu.VMEM_SHARED((N_ACC, DEG_W), jnp.float32),
        ],
    )
    def deg(dst_hbm, ones_hbm, zeros_hbm, out_hbm, dst_v, ones_v, acc):
        c = lax.axis_index("c")
        s = lax.axis_index("s")
        pltpu.sync_copy(dst_hbm.at[c, s], dst_v)
        pltpu.sync_copy(ones_hbm, ones_v)
        pltpu.sync_copy(zeros_hbm, acc.at[pl.ds(s * RPT, RPT)])
        plsc.subcore_barrier()

        def step(j, carry):
            pltpu.sync_copy(ones_v, acc.at[dst_v.at[j]], add=True)
            return carry

        lax.fori_loop(0, lax.select(c == 0, K0, K1), step, 0)
        plsc.subcore_barrier()
        pltpu.sync_copy(acc.at[pl.ds(s * RPT, RPT)],
                        out_hbm.at[c, pl.ds(s * RPT, RPT)])

    return deg


_BLK = 1000


def _dinv_of(degp_ref):
    deg = degp_ref[0, :, :1] + degp_ref[1, :, :1] + 1.0
    return lax.rsqrt(deg)


def _t1_body(degp_ref, feat_ref, w_ref, g_ref):
    dinv = _dinv_of(degp_ref)
    h = jnp.dot(feat_ref[...], w_ref[...], preferred_element_type=jnp.float32)
    g_ref[...] = dinv * h


def _t2_body(degp_ref, aggp_ref, g_ref, b_ref, w_ref, x_ref, gn_ref, *, relu):
    dinv = _dinv_of(degp_ref)
    x = dinv * (aggp_ref[0] + aggp_ref[1] + g_ref[...]) + b_ref[...]
    if relu:
        x = jnp.maximum(x, 0.0)
    x_ref[...] = x
    gn_ref[...] = dinv * jnp.dot(x, w_ref[...],
                                 preferred_element_type=jnp.float32)


def _t3_body(degp_ref, aggp_ref, g_ref, b_ref, x1_ref, x2_ref, wp_ref, bp_ref,
             y_ref):
    dinv = _dinv_of(degp_ref)
    x3 = dinv * (aggp_ref[0] + aggp_ref[1] + g_ref[...]) + b_ref[...]
    wp = wp_ref[...]
    y = jnp.dot(x1_ref[...], wp[0:128], preferred_element_type=jnp.float32)
    y += jnp.dot(x2_ref[...], wp[128:256], preferred_element_type=jnp.float32)
    y += jnp.dot(x3, wp[256:384], preferred_element_type=jnp.float32)
    y_ref[...] = y + bp_ref[...]


def _degp_spec():
    return pl.BlockSpec((NC, _BLK, DEG_W), lambda i: (0, i, 0))


def _aggp_spec():
    return pl.BlockSpec((NC, _BLK, 128), lambda i: (0, i, 0))


def _row_spec(d=128):
    return pl.BlockSpec((_BLK, d), lambda i: (i, 0))


def _full_spec(a, b):
    return pl.BlockSpec((a, b), lambda i: (0, 0))


_GRID = N_NODES // _BLK


def _t1(degp, feat, w1):
    return pl.pallas_call(
        _t1_body,
        grid=(_GRID,),
        in_specs=[_degp_spec(), _row_spec(), _full_spec(128, 128)],
        out_specs=_row_spec(),
        out_shape=jax.ShapeDtypeStruct((N_NODES, 128), jnp.float32),
    )(degp, feat, w1)


def _t2(degp, aggp, g, b, w, relu):
    return pl.pallas_call(
        functools.partial(_t2_body, relu=relu),
        grid=(_GRID,),
        in_specs=[_degp_spec(), _aggp_spec(), _row_spec(),
                  _full_spec(1, 128), _full_spec(128, 128)],
        out_specs=[_row_spec(), _row_spec()],
        out_shape=[jax.ShapeDtypeStruct((N_NODES, 128), jnp.float32),
                   jax.ShapeDtypeStruct((N_NODES, 128), jnp.float32)],
    )(degp, aggp, g, b, w)


def _t3(degp, aggp, g, b, x1, x2, wp, bp):
    return pl.pallas_call(
        _t3_body,
        grid=(_GRID,),
        in_specs=[_degp_spec(), _aggp_spec(), _row_spec(),
                  _full_spec(1, 128), _row_spec(), _row_spec(),
                  _full_spec(384, 16), _full_spec(1, 16)],
        out_specs=_row_spec(16),
        out_shape=jax.ShapeDtypeStruct((N_NODES, 16), jnp.float32),
    )(degp, aggp, g, b, x1, x2, wp, bp)


def kernel(feat, edge_index, batch, W1, b1, W2, b2, W3, b3, Wp, bp):
    del batch
    e = edge_index.shape[1]
    n_chunks = max(K0, K1)
    cap0 = NS * K0 * CHUNK
    pe = cap0 + NS * K1 * CHUNK
    assert pe >= e

    def _split(flat):
        a0 = flat[:cap0].reshape(NS, K0, CHUNK)
        a1 = flat[cap0:].reshape(NS, K1, CHUNK)
        a0 = jnp.pad(a0, ((0, 0), (0, n_chunks - K0), (0, 0)))
        a1 = jnp.pad(a1, ((0, 0), (0, n_chunks - K1), (0, 0)))
        return jnp.stack([a0, a1])

    src = _split(jnp.concatenate(
        [edge_index[0], jnp.zeros((pe - e,), jnp.int32)]))
    dst = _split(jnp.concatenate(
        [edge_index[1], jnp.full((pe - e,), N_NODES, jnp.int32)]))

    zeros_d = jnp.zeros((RPT, 128), jnp.float32)
    zeros_w = jnp.zeros((RPT, DEG_W), jnp.float32)
    ones_w = jnp.ones((CHUNK, DEG_W), jnp.float32)

    agg = _make_agg(n_chunks, 128)
    degp = _make_deg(n_chunks)(dst, ones_w, zeros_w)

    b1r, b2r, b3r, bpr = (x.reshape(1, -1) for x in (b1, b2, b3, bp))

    g1 = _t1(degp, feat, W1)
    a1 = agg(src, dst, g1, zeros_d)
    x1, g2 = _t2(degp, a1, g1, b1r, W2, relu=True)
    a2 = agg(src, dst, g2, zeros_d)
    x2, g3 = _t2(degp, a2, g2, b2r, W3, relu=True)
    a3 = agg(src, dst, g3, zeros_d)
    return _t3(degp, a3, g3, b3r, x1, x2, Wp, bpr)


# speedup vs baseline: 1.0141x; 1.0141x over previous
import functools

import jax
import jax.numpy as jnp
from jax import lax
from jax.experimental import pallas as pl
from jax.experimental.pallas import tpu as pltpu
from jax.experimental.pallas import tpu_sc as plsc

N_NODES = 10000
NC, NS = 2, 16
N_ACC = 10112
RPT = N_ACC // NS
CHUNK = 128
K0 = 80
K1 = 80
DEG_W = 128

_SC_MESH = plsc.VectorSubcoreMesh(core_axis_name="c", subcore_axis_name="s")


NBUF = 2


def _make_agg(n_chunks, d):
    @functools.partial(
        pl.kernel,
        mesh=_SC_MESH,
        out_type=jax.ShapeDtypeStruct((NC, N_ACC, d), jnp.float32),
        scratch_types=[
            pltpu.VMEM((n_chunks, CHUNK), jnp.int32),
            pltpu.VMEM((n_chunks, CHUNK), jnp.int32),
            pltpu.VMEM((CHUNK, d), jnp.float32),
            pltpu.VMEM_SHARED((N_ACC, d), jnp.float32),
        ],
    )
    def agg(src_hbm, dst_hbm, g_hbm, zeros_hbm, out_hbm, src_v, dst_v, rows_v,
            acc):
        c = lax.axis_index("c")
        s = lax.axis_index("s")
        pltpu.sync_copy(src_hbm.at[c, s], src_v)
        pltpu.sync_copy(dst_hbm.at[c, s], dst_v)
        pltpu.sync_copy(zeros_hbm, acc.at[pl.ds(s * RPT, RPT)])
        plsc.subcore_barrier()

        def step(j, carry):
            pltpu.sync_copy(g_hbm.at[src_v.at[j]], rows_v)
            pltpu.sync_copy(rows_v, acc.at[dst_v.at[j]], add=True)
            return carry

        lax.fori_loop(0, n_chunks, step, 0)
        plsc.subcore_barrier()
        pltpu.sync_copy(acc.at[pl.ds(s * RPT, RPT)],
                        out_hbm.at[c, pl.ds(s * RPT, RPT)])

    return agg


def _make_deg(n_chunks):
    @functools.partial(
        pl.kernel,
        mesh=_SC_MESH,
        out_type=jax.ShapeDtypeStruct((NC, N_ACC, DEG_W), jnp.float32),
        scratch_types=[
            pltpu.VMEM((n_chunks, CHUNK), jnp.int32),
            pltpu.VMEM((CHUNK, DEG_W), jnp.float32),
            pltpu.VMEM_SHARED((N_ACC, DEG_W), jnp.float32),
        ],
    )
    def deg(dst_hbm, ones_hbm, zeros_hbm, out_hbm, dst_v, ones_v, acc):
        c = lax.axis_index("c")
        s = lax.axis_index("s")
        pltpu.sync_copy(dst_hbm.at[c, s], dst_v)
        pltpu.sync_copy(ones_hbm, ones_v)
        pltpu.sync_copy(zeros_hbm, acc.at[pl.ds(s * RPT, RPT)])
        plsc.subcore_barrier()

        def step(j, carry):
            pltpu.sync_copy(ones_v, acc.at[dst_v.at[j]], add=True)
            return carry

        lax.fori_loop(0, n_chunks, step, 0)
        plsc.subcore_barrier()
        pltpu.sync_copy(acc.at[pl.ds(s * RPT, RPT)],
                        out_hbm.at[c, pl.ds(s * RPT, RPT)])

    return deg


_BLK = 1000


def _dinv_of(degp_ref):
    deg = degp_ref[0, :, :1] + degp_ref[1, :, :1] + 1.0
    return lax.rsqrt(deg)


def _t1_body(degp_ref, feat_ref, w_ref, g_ref):
    dinv = _dinv_of(degp_ref)
    h = jnp.dot(feat_ref[...], w_ref[...], preferred_element_type=jnp.float32)
    g_ref[...] = dinv * h


def _t2_body(degp_ref, aggp_ref, g_ref, b_ref, w_ref, x_ref, gn_ref, *, relu):
    dinv = _dinv_of(degp_ref)
    x = dinv * (aggp_ref[0] + aggp_ref[1] + g_ref[...]) + b_ref[...]
    if relu:
        x = jnp.maximum(x, 0.0)
    x_ref[...] = x
    gn_ref[...] = dinv * jnp.dot(x, w_ref[...],
                                 preferred_element_type=jnp.float32)


def _t3_body(degp_ref, aggp_ref, g_ref, b_ref, x1_ref, x2_ref, wp_ref, bp_ref,
             y_ref):
    dinv = _dinv_of(degp_ref)
    x3 = dinv * (aggp_ref[0] + aggp_ref[1] + g_ref[...]) + b_ref[...]
    wp = wp_ref[...]
    y = jnp.dot(x1_ref[...], wp[0:128], preferred_element_type=jnp.float32)
    y += jnp.dot(x2_ref[...], wp[128:256], preferred_element_type=jnp.float32)
    y += jnp.dot(x3, wp[256:384], preferred_element_type=jnp.float32)
    y_ref[...] = y + bp_ref[...]


def _degp_spec():
    return pl.BlockSpec((NC, _BLK, DEG_W), lambda i: (0, i, 0))


def _aggp_spec():
    return pl.BlockSpec((NC, _BLK, 128), lambda i: (0, i, 0))


def _row_spec(d=128):
    return pl.BlockSpec((_BLK, d), lambda i: (i, 0))


def _full_spec(a, b):
    return pl.BlockSpec((a, b), lambda i: (0, 0))


_GRID = N_NODES // _BLK


def _t1(degp, feat, w1):
    return pl.pallas_call(
        _t1_body,
        grid=(_GRID,),
        in_specs=[_degp_spec(), _row_spec(), _full_spec(128, 128)],
        out_specs=_row_spec(),
        out_shape=jax.ShapeDtypeStruct((N_NODES, 128), jnp.float32),
    )(degp, feat, w1)


def _t2(degp, aggp, g, b, w, relu):
    return pl.pallas_call(
        functools.partial(_t2_body, relu=relu),
        grid=(_GRID,),
        in_specs=[_degp_spec(), _aggp_spec(), _row_spec(),
                  _full_spec(1, 128), _full_spec(128, 128)],
        out_specs=[_row_spec(), _row_spec()],
        out_shape=[jax.ShapeDtypeStruct((N_NODES, 128), jnp.float32),
                   jax.ShapeDtypeStruct((N_NODES, 128), jnp.float32)],
    )(degp, aggp, g, b, w)


def _t3(degp, aggp, g, b, x1, x2, wp, bp):
    return pl.pallas_call(
        _t3_body,
        grid=(_GRID,),
        in_specs=[_degp_spec(), _aggp_spec(), _row_spec(),
                  _full_spec(1, 128), _row_spec(), _row_spec(),
                  _full_spec(384, 16), _full_spec(1, 16)],
        out_specs=_row_spec(16),
        out_shape=jax.ShapeDtypeStruct((N_NODES, 16), jnp.float32),
    )(degp, aggp, g, b, x1, x2, wp, bp)


def kernel(feat, edge_index, batch, W1, b1, W2, b2, W3, b3, Wp, bp):
    del batch
    e = edge_index.shape[1]
    n_chunks = max(K0, K1)
    cap0 = NS * K0 * CHUNK
    pe = cap0 + NS * K1 * CHUNK
    assert pe >= e

    def _split(flat):
        a0 = flat[:cap0].reshape(NS, K0, CHUNK)
        a1 = flat[cap0:].reshape(NS, K1, CHUNK)
        a0 = jnp.pad(a0, ((0, 0), (0, n_chunks - K0), (0, 0)))
        a1 = jnp.pad(a1, ((0, 0), (0, n_chunks - K1), (0, 0)))
        return jnp.stack([a0, a1])

    src = _split(jnp.concatenate(
        [edge_index[0], jnp.zeros((pe - e,), jnp.int32)]))
    dst = _split(jnp.concatenate(
        [edge_index[1], jnp.full((pe - e,), N_NODES, jnp.int32)]))

    zeros_d = jnp.zeros((RPT, 128), jnp.float32)
    zeros_w = jnp.zeros((RPT, DEG_W), jnp.float32)
    ones_w = jnp.ones((CHUNK, DEG_W), jnp.float32)

    agg = _make_agg(n_chunks, 128)
    degp = _make_deg(n_chunks)(dst, ones_w, zeros_w)

    b1r, b2r, b3r, bpr = (x.reshape(1, -1) for x in (b1, b2, b3, bp))

    g1 = _t1(degp, feat, W1)
    a1 = agg(src, dst, g1, zeros_d)
    x1, g2 = _t2(degp, a1, g1, b1r, W2, relu=True)
    a2 = agg(src, dst, g2, zeros_d)
    x2, g3 = _t2(degp, a2, g2, b2r, W3, relu=True)
    a3 = agg(src, dst, g3, zeros_d)
    return _t3(degp, a3, g3, b3r, x1, x2, Wp, bpr)


# speedup vs baseline: 1.6099x; 1.5876x over previous
import functools

import jax
import jax.numpy as jnp
from jax import lax
from jax.experimental import pallas as pl
from jax.experimental.pallas import tpu as pltpu
from jax.experimental.pallas import tpu_sc as plsc

N_NODES = 10000
NC, NS = 2, 16
N_ACC = 10112
RPT = N_ACC // NS
CHUNK = 128
K0 = 80
K1 = 80
DEG_W = 128

_SC_MESH = plsc.VectorSubcoreMesh(core_axis_name="c", subcore_axis_name="s")


NBUF = 2


def _make_agg(n_chunks, d):
    @functools.partial(
        pl.kernel,
        mesh=_SC_MESH,
        out_type=jax.ShapeDtypeStruct((NC, N_ACC, d), jnp.float32),
        scratch_types=[
            pltpu.VMEM((n_chunks, CHUNK), jnp.int32),
            pltpu.VMEM((n_chunks, CHUNK), jnp.int32),
            pltpu.VMEM((CHUNK, d), jnp.float32),
            pltpu.VMEM_SHARED((N_ACC, d), jnp.float32),
        ],
    )
    def agg(src_hbm, dst_hbm, g_hbm, zeros_hbm, out_hbm, src_v, dst_v, rows_v,
            acc):
        c = lax.axis_index("c")
        s = lax.axis_index("s")
        pltpu.sync_copy(src_hbm.at[c, s], src_v)
        pltpu.sync_copy(dst_hbm.at[c, s], dst_v)
        pltpu.sync_copy(zeros_hbm, acc.at[pl.ds(s * RPT, RPT)])
        plsc.subcore_barrier()

        def step(j, carry):
            pltpu.sync_copy(g_hbm.at[src_v.at[j]], rows_v)
            pltpu.sync_copy(rows_v, acc.at[dst_v.at[j]], add=True)
            return carry

        lax.fori_loop(0, n_chunks, step, 0)
        plsc.subcore_barrier()
        pltpu.sync_copy(acc.at[pl.ds(s * RPT, RPT)],
                        out_hbm.at[c, pl.ds(s * RPT, RPT)])

    return agg


def _make_deg(n_chunks):
    @functools.partial(
        pl.kernel,
        mesh=_SC_MESH,
        out_type=jax.ShapeDtypeStruct((NC, N_ACC, DEG_W), jnp.float32),
        scratch_types=[
            pltpu.VMEM((n_chunks, CHUNK), jnp.int32),
            pltpu.VMEM((CHUNK, DEG_W), jnp.float32),
            pltpu.VMEM_SHARED((N_ACC, DEG_W), jnp.float32),
        ],
    )
    def deg(dst_hbm, ones_hbm, zeros_hbm, out_hbm, dst_v, ones_v, acc):
        c = lax.axis_index("c")
        s = lax.axis_index("s")
        pltpu.sync_copy(dst_hbm.at[c, s], dst_v)
        pltpu.sync_copy(ones_hbm, ones_v)
        pltpu.sync_copy(zeros_hbm, acc.at[pl.ds(s * RPT, RPT)])
        plsc.subcore_barrier()

        def step(j, carry):
            pltpu.sync_copy(ones_v, acc.at[dst_v.at[j]], add=True)
            return carry

        lax.fori_loop(0, n_chunks, step, 0)
        plsc.subcore_barrier()
        pltpu.sync_copy(acc.at[pl.ds(s * RPT, RPT)],
                        out_hbm.at[c, pl.ds(s * RPT, RPT)])

    return deg


_BLK = 1000


def _dinv_of(degp_ref):
    deg = degp_ref[0, :, :1] + degp_ref[1, :, :1] + 1.0
    return lax.rsqrt(deg)


def _t1_body(degp_ref, feat_ref, w_ref, g_ref):
    dinv = _dinv_of(degp_ref)
    h = jnp.dot(feat_ref[...], w_ref[...], preferred_element_type=jnp.float32)
    g_ref[...] = dinv * h


def _t2_body(degp_ref, aggp_ref, g_ref, b_ref, w_ref, x_ref, gn_ref, *, relu):
    dinv = _dinv_of(degp_ref)
    x = dinv * (aggp_ref[0] + aggp_ref[1] + g_ref[...]) + b_ref[...]
    if relu:
        x = jnp.maximum(x, 0.0)
    x_ref[...] = x
    gn_ref[...] = dinv * jnp.dot(x, w_ref[...],
                                 preferred_element_type=jnp.float32)


def _t3_body(degp_ref, aggp_ref, g_ref, b_ref, x1_ref, x2_ref, wp_ref, bp_ref,
             y_ref):
    dinv = _dinv_of(degp_ref)
    x3 = dinv * (aggp_ref[0] + aggp_ref[1] + g_ref[...]) + b_ref[...]
    wp = wp_ref[...]
    y = jnp.dot(x1_ref[...], wp[0:128], preferred_element_type=jnp.float32)
    y += jnp.dot(x2_ref[...], wp[128:256], preferred_element_type=jnp.float32)
    y += jnp.dot(x3, wp[256:384], preferred_element_type=jnp.float32)
    y_ref[...] = y + bp_ref[...]


def _degp_spec():
    return pl.BlockSpec((NC, _BLK, DEG_W), lambda i: (0, i, 0))


def _aggp_spec():
    return pl.BlockSpec((NC, _BLK, 128), lambda i: (0, i, 0))


def _row_spec(d=128):
    return pl.BlockSpec((_BLK, d), lambda i: (i, 0))


def _full_spec(a, b):
    return pl.BlockSpec((a, b), lambda i: (0, 0))


_GRID = N_NODES // _BLK


def _t1(degp, feat, w1):
    return pl.pallas_call(
        _t1_body,
        grid=(_GRID,),
        in_specs=[_degp_spec(), _row_spec(), _full_spec(128, 128)],
        out_specs=_row_spec(),
        out_shape=jax.ShapeDtypeStruct((N_NODES, 128), jnp.float32),
    )(degp, feat, w1)


def _t2(degp, aggp, g, b, w, relu):
    return pl.pallas_call(
        functools.partial(_t2_body, relu=relu),
        grid=(_GRID,),
        in_specs=[_degp_spec(), _aggp_spec(), _row_spec(),
                  _full_spec(1, 128), _full_spec(128, 128)],
        out_specs=[_row_spec(), _row_spec()],
        out_shape=[jax.ShapeDtypeStruct((N_NODES, 128), jnp.float32),
                   jax.ShapeDtypeStruct((N_NODES, 128), jnp.float32)],
    )(degp, aggp, g, b, w)


def _t3(degp, aggp, g, b, x1, x2, wp, bp):
    return pl.pallas_call(
        _t3_body,
        grid=(_GRID,),
        in_specs=[_degp_spec(), _aggp_spec(), _row_spec(),
                  _full_spec(1, 128), _row_spec(), _row_spec(),
                  _full_spec(384, 16), _full_spec(1, 16)],
        out_specs=_row_spec(16),
        out_shape=jax.ShapeDtypeStruct((N_NODES, 16), jnp.float32),
    )(degp, aggp, g, b, x1, x2, wp, bp)


def kernel(feat, edge_index, batch, W1, b1, W2, b2, W3, b3, Wp, bp):
    del batch
    e = edge_index.shape[1]
    per_tile = -(-e // (NC * NS * CHUNK)) * CHUNK
    n_chunks = per_tile // CHUNK
    pe = NC * NS * per_tile
    src = jnp.concatenate(
        [edge_index[0], jnp.zeros((pe - e,), jnp.int32)]).reshape(
            NC, NS, n_chunks, CHUNK)
    dst = jnp.concatenate(
        [edge_index[1], jnp.full((pe - e,), N_NODES, jnp.int32)]).reshape(
            NC, NS, n_chunks, CHUNK)

    zeros_d = jnp.zeros((RPT, 128), jnp.float32)
    zeros_w = jnp.zeros((RPT, DEG_W), jnp.float32)
    ones_w = jnp.ones((CHUNK, DEG_W), jnp.float32)

    agg = _make_agg(n_chunks, 128)
    degp = _make_deg(n_chunks)(dst, ones_w, zeros_w)

    b1r, b2r, b3r, bpr = (x.reshape(1, -1) for x in (b1, b2, b3, bp))

    g1 = _t1(degp, feat, W1)
    a1 = agg(src, dst, g1, zeros_d)
    x1, g2 = _t2(degp, a1, g1, b1r, W2, relu=True)
    a2 = agg(src, dst, g2, zeros_d)
    x2, g3 = _t2(degp, a2, g2, b2r, W3, relu=True)
    a3 = agg(src, dst, g3, zeros_d)
    return _t3(degp, a3, g3, b3r, x1, x2, Wp, bpr)
